# baseline (speedup 1.0000x reference)
import functools

import jax
import jax.numpy as jnp
from jax import lax
from jax.experimental import pallas as pl
from jax.experimental.pallas import tpu as pltpu
from jax.experimental.pallas import tpu_sc as plsc

N = 10000
E = 160000
D = 256
DH = 128
G = 128
NC = 40
BN_EPS = 1e-5

CHUNK = 50
GRP = 4
NGRP = E // (CHUNK * GRP)
GRP_PT = NGRP // 16
NBUF = 4
ROWS_PT = 624
ROWS_LAST = N - 15 * ROWS_PT
SEG_CH = 208

_mesh = plsc.VectorSubcoreMesh(core_axis_name="c", subcore_axis_name="s",
                               num_cores=2, num_subcores=16)


def _scatter_body(x_hbm, src_hbm, dst_hbm, out_hbm,
                  src_v, dst_v, rows_v, acc_s, gsem, ssem, isem):
    c = lax.axis_index("c")
    s = lax.axis_index("s")

    pltpu.sync_copy(x_hbm.at[c].at[pl.ds(s * ROWS_PT, ROWS_PT)],
                    acc_s.at[pl.ds(s * ROWS_PT, ROWS_PT)])

    @pl.when(s == 15)
    def _():
        pltpu.sync_copy(x_hbm.at[c].at[pl.ds(16 * ROWS_PT, ROWS_LAST - ROWS_PT)],
                        acc_s.at[pl.ds(16 * ROWS_PT, ROWS_LAST - ROWS_PT)])

    plsc.subcore_barrier()

    xh = x_hbm.at[c]
    gbase = s * GRP_PT

    def load_idx(g, a):
        pltpu.async_copy(src_hbm.at[gbase + g], src_v.at[a], isem.at[a])
        pltpu.async_copy(dst_hbm.at[gbase + g], dst_v.at[a], isem.at[a])

    def wait_idx(a):
        pltpu.make_async_copy(src_hbm.at[0], src_v.at[a], isem.at[a]).wait()
        pltpu.make_async_copy(dst_hbm.at[0], dst_v.at[a], isem.at[a]).wait()

    def gather(a, k, b):
        pltpu.async_copy(xh.at[src_v.at[a].at[k]], rows_v.at[b], gsem.at[b])

    def wait_gather(b):
        pltpu.make_async_copy(xh.at[src_v.at[0].at[0]], rows_v.at[b],
                              gsem.at[b]).wait()

    def scatter(a, k):
        pltpu.async_copy(rows_v.at[k], acc_s.at[dst_v.at[a].at[k]],
                         ssem.at[k], add=True)

    def wait_scatter(b):
        pltpu.make_async_copy(rows_v.at[b], acc_s.at[dst_v.at[0].at[0]],
                              ssem.at[b]).wait()

    load_idx(0, 0)
    load_idx(1, 1)
    wait_idx(0)
    gather(0, 0, 0)
    gather(0, 1, 1)

    def group_body(g, aa, first, last, traced):
        for k in range(GRP):
            wait_gather(k)
            scatter(aa, k)
            nb = (k + 2) % NBUF
            if k < 2:
                if not first:
                    wait_scatter(nb)
                gather(aa, k + 2, nb)
            elif not last:
                if k == 2:
                    wait_idx(1 - aa)
                wait_scatter(nb)
                gather(1 - aa, k - 2, nb)
        if not last:
            if traced:
                @pl.when(g + 2 < GRP_PT)
                def _():
                    load_idx(g + 2, aa)
            elif g + 2 < GRP_PT:
                load_idx(g + 2, aa)

    group_body(0, 0, True, False, False)

    def gpair(t, carry):
        g0 = 1 + 2 * t
        group_body(g0, 1, False, False, True)
        group_body(g0 + 1, 0, False, False, True)
        return carry

    lax.fori_loop(0, (GRP_PT - 2) // 2, gpair, 0)
    group_body(GRP_PT - 1, 1, False, True, False)

    for b in range(NBUF):
        wait_scatter(b)

    plsc.subcore_barrier()

    pltpu.sync_copy(acc_s.at[pl.ds(s * ROWS_PT, ROWS_PT)],
                    out_hbm.at[c].at[pl.ds(s * ROWS_PT, ROWS_PT)])

    @pl.when(s == 15)
    def _():
        pltpu.sync_copy(acc_s.at[pl.ds(16 * ROWS_PT, ROWS_LAST - ROWS_PT)],
                        out_hbm.at[c].at[pl.ds(16 * ROWS_PT, ROWS_LAST - ROWS_PT)])


_scatter = functools.partial(
    pl.kernel, _scatter_body,
    out_type=jax.ShapeDtypeStruct((2, N, DH), jnp.float32),
    mesh=_mesh,
    scratch_types=[
        pltpu.VMEM((2, GRP, CHUNK), jnp.int32),
        pltpu.VMEM((2, GRP, CHUNK), jnp.int32),
        pltpu.VMEM((NBUF, CHUNK, DH), jnp.float32),
        pltpu.VMEM_SHARED((N, DH), jnp.float32),
        pltpu.SemaphoreType.DMA((NBUF,)),
        pltpu.SemaphoreType.DMA((NBUF,)),
        pltpu.SemaphoreType.DMA((2,)),
    ],
)()


NB = 10
BR = N // NB


def _mlp_body(h_ref, Wa_ref, ba_ref, Wb_ref, bb_ref, g_ref, be_ref,
              o_ref, y_scr, st_scr):
    p = pl.program_id(0)
    i = pl.program_id(1)

    @pl.when(p == 0)
    def _():
        hb = h_ref[...]
        h = jnp.concatenate([hb[0], hb[1]], axis=1)
        t = jnp.maximum(
            jnp.dot(h, Wa_ref[...], preferred_element_type=jnp.float32)
            + ba_ref[...], 0.0)
        y = jnp.maximum(
            jnp.dot(t, Wb_ref[...], preferred_element_type=jnp.float32)
            + bb_ref[...], 0.0)
        y_scr[pl.ds(i * BR, BR), :] = y

        @pl.when(i == 0)
        def _():
            st_scr[...] = jnp.zeros_like(st_scr)

        st_scr[0:1, :] += jnp.sum(y, axis=0, keepdims=True)
        st_scr[1:2, :] += jnp.sum(y * y, axis=0, keepdims=True)

    @pl.when(p == 1)
    def _():
        mu = st_scr[0:1, :] * (1.0 / N)
        var = st_scr[1:2, :] * (1.0 / N) - mu * mu
        scale = g_ref[...] * jax.lax.rsqrt(var + BN_EPS)
        shift = be_ref[...] - mu * scale
        y = y_scr[pl.ds(i * BR, BR), :] * scale + shift
        o_ref[0, :, :] = y[:, :DH]
        o_ref[1, :, :] = y[:, DH:]


def _mlp(h, Wa, ba, Wb, bb, g, be):
    wspec = pl.BlockSpec((D, D), lambda p, i: (0, 0))
    vspec = pl.BlockSpec((1, D), lambda p, i: (0, 0))
    return pl.pallas_call(
        _mlp_body,
        grid=(2, NB),
        in_specs=[
            pl.BlockSpec((2, BR, DH), lambda p, i: (0, jnp.where(p == 0, i, 0), 0)),
            wspec, vspec, wspec, vspec, vspec, vspec,
        ],
        out_specs=pl.BlockSpec((2, BR, DH), lambda p, i: (0, jnp.where(p == 1, i, 0), 0)),
        out_shape=jax.ShapeDtypeStruct((2, N, DH), jnp.float32),
        scratch_shapes=[
            pltpu.VMEM((N, D), jnp.float32),
            pltpu.VMEM((8, D), jnp.float32),
        ],
    )(h, Wa, ba, Wb, bb, g, be)


def _segmax_body(x_hbm, bat_hbm, neg_hbm, out_hbm, acc_v, rows_v, bat_v):
    c = lax.axis_index("c")
    s = lax.axis_index("s")

    pltpu.sync_copy(neg_hbm, acc_v)
    pltpu.sync_copy(bat_hbm.at[pl.ds(s * ROWS_PT, ROWS_PT)],
                    bat_v.at[pl.ds(0, ROWS_PT)])

    @pl.when(s == 15)
    def _():
        pltpu.sync_copy(bat_hbm.at[pl.ds(16 * ROWS_PT, ROWS_LAST - ROWS_PT)],
                        bat_v.at[pl.ds(ROWS_PT, ROWS_LAST - ROWS_PT)])

    xh = x_hbm.at[c]

    def do_rows(local_base, nrows):
        def row(r, carry2):
            gidx = bat_v[pl.ds(local_base + r, 16)][0]
            for k in range(DH // 16):
                sl = pl.ds(k * 16, 16)
                acc_v[gidx, sl] = jnp.maximum(acc_v[gidx, sl], rows_v[r, sl])
            return carry2

        lax.fori_loop(0, nrows, row, 0)

    def chunk(cb, carry):
        pltpu.sync_copy(xh.at[pl.ds(s * ROWS_PT + cb * SEG_CH, SEG_CH)], rows_v)
        do_rows(cb * SEG_CH, SEG_CH)
        return carry

    lax.fori_loop(0, ROWS_PT // SEG_CH, chunk, 0)

    @pl.when(s == 15)
    def _():
        pltpu.sync_copy(xh.at[pl.ds(15 * ROWS_PT + ROWS_PT, ROWS_LAST - ROWS_PT)],
                        rows_v.at[pl.ds(0, ROWS_LAST - ROWS_PT)])
        do_rows(ROWS_PT, ROWS_LAST - ROWS_PT)

    pltpu.sync_copy(acc_v, out_hbm.at[c].at[s])


_segmax = functools.partial(
    pl.kernel, _segmax_body,
    out_type=jax.ShapeDtypeStruct((2, 16, G, DH), jnp.float32),
    mesh=_mesh,
    scratch_types=[
        pltpu.VMEM((G, DH), jnp.float32),
        pltpu.VMEM((SEG_CH, DH), jnp.float32),
        pltpu.VMEM((ROWS_LAST + 16, ), jnp.int32),
    ],
)()


def _final_body(p_ref, Wf_ref, bf_ref, o_ref):
    red0 = p_ref[0, 0]
    red1 = p_ref[1, 0]
    for s in range(1, 16):
        red0 = jnp.maximum(red0, p_ref[0, s])
        red1 = jnp.maximum(red1, p_ref[1, s])
    pooled = jnp.concatenate([red0, red1], axis=1)
    z = jnp.dot(pooled, Wf_ref[...], preferred_element_type=jnp.float32) \
        + bf_ref[...]
    o_ref[...] = 1.0 / (1.0 + jnp.exp(-z))


def _final(parts, Wf, bf):
    return pl.pallas_call(
        _final_body,
        out_shape=jax.ShapeDtypeStruct((G, NC), jnp.float32),
    )(parts, Wf, bf)


def kernel(data_base, edge_index_base, batch_base,
           W1a, b1a, W1b, b1b, W2a, b2a, W2b, b2b, W3a, b3a, W3b, b3b,
           g1, be1, g2, be2, g3, be3, Wf, bf):
    x = data_base.reshape(N, 2, DH).transpose(1, 0, 2)
    src2d = edge_index_base[0].reshape(NGRP, GRP, CHUNK)
    dst2d = edge_index_base[1].reshape(NGRP, GRP, CHUNK)
    neg = jnp.full((G, DH), -jnp.inf, jnp.float32)

    layers = [(W1a, b1a, W1b, b1b, g1, be1),
              (W2a, b2a, W2b, b2b, g2, be2),
              (W3a, b3a, W3b, b3b, g3, be3)]
    for Wa, ba, Wb, bb, g, be in layers:
        h = _scatter(x, src2d, dst2d)
        x = _mlp(h, Wa, ba.reshape(1, D), Wb, bb.reshape(1, D),
                 g.reshape(1, D), be.reshape(1, D))

    parts = _segmax(x, batch_base, neg)
    return _final(parts, Wf, bf.reshape(1, NC))


# speedup vs baseline: 1.1823x; 1.1823x over previous
import functools

import jax
import jax.numpy as jnp
from jax import lax
from jax.experimental import pallas as pl
from jax.experimental.pallas import tpu as pltpu
from jax.experimental.pallas import tpu_sc as plsc

N = 10000
E = 160000
D = 256
DH = 128
G = 128
NC = 40
BN_EPS = 1e-5

CHUNK = 125
GRP = 8
NGRP = E // (CHUNK * GRP)
GRP_PT = NGRP // 16
CH_T = GRP_PT * GRP
ROWS_PT = 624
ROWS_LAST = N - 15 * ROWS_PT
SEG_CH = 208

_mesh = plsc.VectorSubcoreMesh(core_axis_name="c", subcore_axis_name="s",
                               num_cores=2, num_subcores=16)


def _scatter_body(x_hbm, src_hbm, dst_hbm, out_hbm,
                  src_v, dst_v, rows_v, acc_s, gsem, isem):
    c = lax.axis_index("c")
    s = lax.axis_index("s")

    pltpu.sync_copy(x_hbm.at[c].at[pl.ds(s * ROWS_PT, ROWS_PT)],
                    acc_s.at[pl.ds(s * ROWS_PT, ROWS_PT)])

    @pl.when(s == 15)
    def _():
        pltpu.sync_copy(x_hbm.at[c].at[pl.ds(16 * ROWS_PT, ROWS_LAST - ROWS_PT)],
                        acc_s.at[pl.ds(16 * ROWS_PT, ROWS_LAST - ROWS_PT)])

    plsc.subcore_barrier()

    xh = x_hbm.at[c]
    gbase = s * GRP_PT

    def load_idx(g, a):
        pltpu.async_copy(src_hbm.at[gbase + g], src_v.at[a], isem.at[a])
        pltpu.async_copy(dst_hbm.at[gbase + g], dst_v.at[a], isem.at[a])

    def wait_idx(a):
        pltpu.make_async_copy(src_hbm.at[0], src_v.at[a], isem.at[a]).wait()
        pltpu.make_async_copy(dst_hbm.at[0], dst_v.at[a], isem.at[a]).wait()

    def gather(a, k, b):
        pltpu.async_copy(xh.at[src_v.at[a].at[k]], rows_v.at[b], gsem.at[b])

    def wait_gather(b):
        pltpu.make_async_copy(xh.at[src_v.at[0].at[0]], rows_v.at[b],
                              gsem.at[b]).wait()

    def scatter(a, k, b):
        pltpu.sync_copy(rows_v.at[b], acc_s.at[dst_v.at[a].at[k]], add=True)

    load_idx(0, 0)
    load_idx(1, 1)
    wait_idx(0)
    gather(0, 0, 0)
    gather(0, 1, 1)

    def group_body(g, aa, last, traced):
        for k in range(GRP):
            b = k % 2
            wait_gather(b)
            scatter(aa, k, b)
            if k < GRP - 2:
                gather(aa, k + 2, b)
            elif not last:
                if k == GRP - 2:
                    wait_idx(1 - aa)
                gather(1 - aa, k + 2 - GRP, b)
        if not last:
            if traced:
                @pl.when(g + 2 < GRP_PT)
                def _():
                    load_idx(g + 2, aa)
            elif g + 2 < GRP_PT:
                load_idx(g + 2, aa)

    def gpair(t, carry):
        g0 = 2 * t
        for aa in range(2):
            group_body(g0 + aa, aa, False, True)
        return carry

    if GRP_PT % 2 == 1:
        lax.fori_loop(0, (GRP_PT - 1) // 2, gpair, 0)
        group_body(GRP_PT - 1, 0, True, False)
    else:
        lax.fori_loop(0, (GRP_PT - 2) // 2, gpair, 0)
        group_body(GRP_PT - 2, 0, False, False)
        group_body(GRP_PT - 1, 1, True, False)

    plsc.subcore_barrier()

    pltpu.sync_copy(acc_s.at[pl.ds(s * ROWS_PT, ROWS_PT)],
                    out_hbm.at[c].at[pl.ds(s * ROWS_PT, ROWS_PT)])

    @pl.when(s == 15)
    def _():
        pltpu.sync_copy(acc_s.at[pl.ds(16 * ROWS_PT, ROWS_LAST - ROWS_PT)],
                        out_hbm.at[c].at[pl.ds(16 * ROWS_PT, ROWS_LAST - ROWS_PT)])


_scatter = functools.partial(
    pl.kernel, _scatter_body,
    out_type=jax.ShapeDtypeStruct((2, N, DH), jnp.float32),
    mesh=_mesh,
    scratch_types=[
        pltpu.VMEM((2, GRP, CHUNK), jnp.int32),
        pltpu.VMEM((2, GRP, CHUNK), jnp.int32),
        pltpu.VMEM((2, CHUNK, DH), jnp.float32),
        pltpu.VMEM_SHARED((N, DH), jnp.float32),
        pltpu.SemaphoreType.DMA((2,)),
        pltpu.SemaphoreType.DMA((2,)),
    ],
)()


NB = 5
BR = N // NB


def _mlp_body(h_ref, Wa_ref, ba_ref, Wb_ref, bb_ref, g_ref, be_ref,
              o_ref, y_scr, st_scr):
    p = pl.program_id(0)
    i = pl.program_id(1)

    @pl.when(p == 0)
    def _():
        hb = h_ref[...]
        h = jnp.concatenate([hb[0], hb[1]], axis=1)
        t = jnp.maximum(
            jnp.dot(h, Wa_ref[...], preferred_element_type=jnp.float32)
            + ba_ref[...], 0.0)
        y = jnp.maximum(
            jnp.dot(t, Wb_ref[...], preferred_element_type=jnp.float32)
            + bb_ref[...], 0.0)
        y_scr[pl.ds(i * BR, BR), :] = y

        @pl.when(i == 0)
        def _():
            st_scr[...] = jnp.zeros_like(st_scr)

        st_scr[0:1, :] += jnp.sum(y, axis=0, keepdims=True)
        st_scr[1:2, :] += jnp.sum(y * y, axis=0, keepdims=True)

    @pl.when(p == 1)
    def _():
        mu = st_scr[0:1, :] * (1.0 / N)
        var = st_scr[1:2, :] * (1.0 / N) - mu * mu
        scale = g_ref[...] * jax.lax.rsqrt(var + BN_EPS)
        shift = be_ref[...] - mu * scale
        y = y_scr[pl.ds(i * BR, BR), :] * scale + shift
        o_ref[0, :, :] = y[:, :DH]
        o_ref[1, :, :] = y[:, DH:]


def _mlp(h, Wa, ba, Wb, bb, g, be):
    wspec = pl.BlockSpec((D, D), lambda p, i: (0, 0))
    vspec = pl.BlockSpec((1, D), lambda p, i: (0, 0))
    return pl.pallas_call(
        _mlp_body,
        grid=(2, NB),
        in_specs=[
            pl.BlockSpec((2, BR, DH), lambda p, i: (0, jnp.where(p == 0, i, 0), 0)),
            wspec, vspec, wspec, vspec, vspec, vspec,
        ],
        out_specs=pl.BlockSpec((2, BR, DH), lambda p, i: (0, jnp.where(p == 1, i, 0), 0)),
        out_shape=jax.ShapeDtypeStruct((2, N, DH), jnp.float32),
        scratch_shapes=[
            pltpu.VMEM((N, D), jnp.float32),
            pltpu.VMEM((8, D), jnp.float32),
        ],
    )(h, Wa, ba, Wb, bb, g, be)


def _segmax_body(x_hbm, bat_hbm, neg_hbm, out_hbm, acc_v, rows_v, bat_v):
    c = lax.axis_index("c")
    s = lax.axis_index("s")

    pltpu.sync_copy(neg_hbm, acc_v)
    pltpu.sync_copy(bat_hbm.at[pl.ds(s * ROWS_PT, ROWS_PT)],
                    bat_v.at[pl.ds(0, ROWS_PT)])

    @pl.when(s == 15)
    def _():
        pltpu.sync_copy(bat_hbm.at[pl.ds(16 * ROWS_PT, ROWS_LAST - ROWS_PT)],
                        bat_v.at[pl.ds(ROWS_PT, ROWS_LAST - ROWS_PT)])

    xh = x_hbm.at[c]

    def do_rows(local_base, nrows):
        def row(r, carry2):
            gidx = bat_v[pl.ds(local_base + r, 16)][0]
            for k in range(DH // 16):
                sl = pl.ds(k * 16, 16)
                acc_v[gidx, sl] = jnp.maximum(acc_v[gidx, sl], rows_v[r, sl])
            return carry2

        lax.fori_loop(0, nrows, row, 0)

    def chunk(cb, carry):
        pltpu.sync_copy(xh.at[pl.ds(s * ROWS_PT + cb * SEG_CH, SEG_CH)], rows_v)
        do_rows(cb * SEG_CH, SEG_CH)
        return carry

    lax.fori_loop(0, ROWS_PT // SEG_CH, chunk, 0)

    @pl.when(s == 15)
    def _():
        pltpu.sync_copy(xh.at[pl.ds(15 * ROWS_PT + ROWS_PT, ROWS_LAST - ROWS_PT)],
                        rows_v.at[pl.ds(0, ROWS_LAST - ROWS_PT)])
        do_rows(ROWS_PT, ROWS_LAST - ROWS_PT)

    pltpu.sync_copy(acc_v, out_hbm.at[c].at[s])


_segmax = functools.partial(
    pl.kernel, _segmax_body,
    out_type=jax.ShapeDtypeStruct((2, 16, G, DH), jnp.float32),
    mesh=_mesh,
    scratch_types=[
        pltpu.VMEM((G, DH), jnp.float32),
        pltpu.VMEM((SEG_CH, DH), jnp.float32),
        pltpu.VMEM((ROWS_LAST + 16, ), jnp.int32),
    ],
)()


def _final_body(p_ref, Wf_ref, bf_ref, o_ref):
    red0 = p_ref[0, 0]
    red1 = p_ref[1, 0]
    for s in range(1, 16):
        red0 = jnp.maximum(red0, p_ref[0, s])
        red1 = jnp.maximum(red1, p_ref[1, s])
    pooled = jnp.concatenate([red0, red1], axis=1)
    z = jnp.dot(pooled, Wf_ref[...], preferred_element_type=jnp.float32) \
        + bf_ref[...]
    o_ref[...] = 1.0 / (1.0 + jnp.exp(-z))


def _final(parts, Wf, bf):
    return pl.pallas_call(
        _final_body,
        out_shape=jax.ShapeDtypeStruct((G, NC), jnp.float32),
    )(parts, Wf, bf)


def kernel(data_base, edge_index_base, batch_base,
           W1a, b1a, W1b, b1b, W2a, b2a, W2b, b2b, W3a, b3a, W3b, b3b,
           g1, be1, g2, be2, g3, be3, Wf, bf):
    x = data_base.reshape(N, 2, DH).transpose(1, 0, 2)
    src2d = edge_index_base[0].reshape(NGRP, GRP, CHUNK)
    dst2d = edge_index_base[1].reshape(NGRP, GRP, CHUNK)
    neg = jnp.full((G, DH), -jnp.inf, jnp.float32)

    layers = [(W1a, b1a, W1b, b1b, g1, be1),
              (W2a, b2a, W2b, b2b, g2, be2),
              (W3a, b3a, W3b, b3b, g3, be3)]
    for Wa, ba, Wb, bb, g, be in layers:
        h = _scatter(x, src2d, dst2d)
        x = _mlp(h, Wa, ba.reshape(1, D), Wb, bb.reshape(1, D),
                 g.reshape(1, D), be.reshape(1, D))

    parts = _segmax(x, batch_base, neg)
    return _final(parts, Wf, bf.reshape(1, NC))


# speedup vs baseline: 1.1906x; 1.0070x over previous
import functools

import jax
import jax.numpy as jnp
from jax import lax
from jax.experimental import pallas as pl
from jax.experimental.pallas import tpu as pltpu
from jax.experimental.pallas import tpu_sc as plsc

N = 10000
E = 160000
D = 256
DH = 128
G = 128
NC = 40
BN_EPS = 1e-5

CHUNK = 125
GRP = 8
NGRP = E // (CHUNK * GRP)
GRP_PT = NGRP // 16
CH_T = GRP_PT * GRP
ROWS_PT = 624
ROWS_LAST = N - 15 * ROWS_PT
SEG_CH = 208

_mesh = plsc.VectorSubcoreMesh(core_axis_name="c", subcore_axis_name="s",
                               num_cores=2, num_subcores=16)


def _scatter_body(x_hbm, src_hbm, dst_hbm, out_hbm,
                  src_v, dst_v, rows_v, acc_s, gsem, isem):
    c = lax.axis_index("c")
    s = lax.axis_index("s")

    pltpu.sync_copy(x_hbm.at[c].at[pl.ds(s * ROWS_PT, ROWS_PT)],
                    acc_s.at[pl.ds(s * ROWS_PT, ROWS_PT)])

    @pl.when(s == 15)
    def _():
        pltpu.sync_copy(x_hbm.at[c].at[pl.ds(16 * ROWS_PT, ROWS_LAST - ROWS_PT)],
                        acc_s.at[pl.ds(16 * ROWS_PT, ROWS_LAST - ROWS_PT)])

    plsc.subcore_barrier()

    xh = x_hbm.at[c]
    gbase = s * GRP_PT

    def load_idx(g, a):
        pltpu.async_copy(src_hbm.at[gbase + g], src_v.at[a], isem.at[a])
        pltpu.async_copy(dst_hbm.at[gbase + g], dst_v.at[a], isem.at[a])

    def wait_idx(a):
        pltpu.make_async_copy(src_hbm.at[0], src_v.at[a], isem.at[a]).wait()
        pltpu.make_async_copy(dst_hbm.at[0], dst_v.at[a], isem.at[a]).wait()

    def gather(a, k, b):
        pltpu.async_copy(xh.at[src_v.at[a].at[k]], rows_v.at[b], gsem.at[b])

    def wait_gather(b):
        pltpu.make_async_copy(xh.at[src_v.at[0].at[0]], rows_v.at[b],
                              gsem.at[b]).wait()

    def scatter(a, k, b):
        pltpu.sync_copy(rows_v.at[b], acc_s.at[dst_v.at[a].at[k]], add=True)

    load_idx(0, 0)
    load_idx(1, 1)
    wait_idx(0)
    gather(0, 0, 0)
    gather(0, 1, 1)

    def group_body(g, aa, last, traced):
        for k in range(GRP):
            b = k % 2
            wait_gather(b)
            scatter(aa, k, b)
            if k < GRP - 2:
                gather(aa, k + 2, b)
            elif not last:
                if k == GRP - 2:
                    wait_idx(1 - aa)
                gather(1 - aa, k + 2 - GRP, b)
        if not last:
            if traced:
                @pl.when(g + 2 < GRP_PT)
                def _():
                    load_idx(g + 2, aa)
            elif g + 2 < GRP_PT:
                load_idx(g + 2, aa)

    def gpair(t, carry):
        g0 = 2 * t
        for aa in range(2):
            group_body(g0 + aa, aa, False, True)
        return carry

    if GRP_PT % 2 == 1:
        lax.fori_loop(0, (GRP_PT - 1) // 2, gpair, 0)
        group_body(GRP_PT - 1, 0, True, False)
    else:
        lax.fori_loop(0, (GRP_PT - 2) // 2, gpair, 0)
        group_body(GRP_PT - 2, 0, False, False)
        group_body(GRP_PT - 1, 1, True, False)

    plsc.subcore_barrier()

    pltpu.sync_copy(acc_s.at[pl.ds(s * ROWS_PT, ROWS_PT)],
                    out_hbm.at[c].at[pl.ds(s * ROWS_PT, ROWS_PT)])

    @pl.when(s == 15)
    def _():
        pltpu.sync_copy(acc_s.at[pl.ds(16 * ROWS_PT, ROWS_LAST - ROWS_PT)],
                        out_hbm.at[c].at[pl.ds(16 * ROWS_PT, ROWS_LAST - ROWS_PT)])


_scatter = functools.partial(
    pl.kernel, _scatter_body,
    out_type=jax.ShapeDtypeStruct((2, N, DH), jnp.float32),
    mesh=_mesh,
    scratch_types=[
        pltpu.VMEM((2, GRP, CHUNK), jnp.int32),
        pltpu.VMEM((2, GRP, CHUNK), jnp.int32),
        pltpu.VMEM((2, CHUNK, DH), jnp.float32),
        pltpu.VMEM_SHARED((N, DH), jnp.float32),
        pltpu.SemaphoreType.DMA((2,)),
        pltpu.SemaphoreType.DMA((2,)),
    ],
)()


NB = 2
BR = N // NB


def _mlp_body(h_ref, Wa_ref, ba_ref, Wb_ref, bb_ref, g_ref, be_ref,
              o_ref, y_scr, st_scr):
    p = pl.program_id(0)
    i = pl.program_id(1)

    @pl.when(p == 0)
    def _():
        hb = h_ref[...]
        h = jnp.concatenate([hb[0], hb[1]], axis=1)
        t = jnp.maximum(
            jnp.dot(h, Wa_ref[...], preferred_element_type=jnp.float32)
            + ba_ref[...], 0.0)
        y = jnp.maximum(
            jnp.dot(t, Wb_ref[...], preferred_element_type=jnp.float32)
            + bb_ref[...], 0.0)
        y_scr[pl.ds(i * BR, BR), :] = y

        @pl.when(i == 0)
        def _():
            st_scr[...] = jnp.zeros_like(st_scr)

        st_scr[0:1, :] += jnp.sum(y, axis=0, keepdims=True)
        st_scr[1:2, :] += jnp.sum(y * y, axis=0, keepdims=True)

    @pl.when(p == 1)
    def _():
        mu = st_scr[0:1, :] * (1.0 / N)
        var = st_scr[1:2, :] * (1.0 / N) - mu * mu
        scale = g_ref[...] * jax.lax.rsqrt(var + BN_EPS)
        shift = be_ref[...] - mu * scale
        y = y_scr[pl.ds(i * BR, BR), :] * scale + shift
        o_ref[0, :, :] = y[:, :DH]
        o_ref[1, :, :] = y[:, DH:]


def _mlp(h, Wa, ba, Wb, bb, g, be):
    wspec = pl.BlockSpec((D, D), lambda p, i: (0, 0))
    vspec = pl.BlockSpec((1, D), lambda p, i: (0, 0))
    return pl.pallas_call(
        _mlp_body,
        grid=(2, NB),
        in_specs=[
            pl.BlockSpec((2, BR, DH), lambda p, i: (0, jnp.where(p == 0, i, 0), 0)),
            wspec, vspec, wspec, vspec, vspec, vspec,
        ],
        out_specs=pl.BlockSpec((2, BR, DH), lambda p, i: (0, jnp.where(p == 1, i, 0), 0)),
        out_shape=jax.ShapeDtypeStruct((2, N, DH), jnp.float32),
        scratch_shapes=[
            pltpu.VMEM((N, D), jnp.float32),
            pltpu.VMEM((8, D), jnp.float32),
        ],
    )(h, Wa, ba, Wb, bb, g, be)


def _segmax_body(x_hbm, bat_hbm, neg_hbm, out_hbm, acc_v, rows_v, bat_v):
    c = lax.axis_index("c")
    s = lax.axis_index("s")

    pltpu.sync_copy(neg_hbm, acc_v)
    pltpu.sync_copy(bat_hbm.at[pl.ds(s * ROWS_PT, ROWS_PT)],
                    bat_v.at[pl.ds(0, ROWS_PT)])

    @pl.when(s == 15)
    def _():
        pltpu.sync_copy(bat_hbm.at[pl.ds(16 * ROWS_PT, ROWS_LAST - ROWS_PT)],
                        bat_v.at[pl.ds(ROWS_PT, ROWS_LAST - ROWS_PT)])

    xh = x_hbm.at[c]

    def do_rows(local_base, nrows):
        def row(r, carry2):
            gidx = bat_v[pl.ds(local_base + r, 16)][0]
            for k in range(DH // 16):
                sl = pl.ds(k * 16, 16)
                acc_v[gidx, sl] = jnp.maximum(acc_v[gidx, sl], rows_v[r, sl])
            return carry2

        lax.fori_loop(0, nrows, row, 0)

    def chunk(cb, carry):
        pltpu.sync_copy(xh.at[pl.ds(s * ROWS_PT + cb * SEG_CH, SEG_CH)], rows_v)
        do_rows(cb * SEG_CH, SEG_CH)
        return carry

    lax.fori_loop(0, ROWS_PT // SEG_CH, chunk, 0)

    @pl.when(s == 15)
    def _():
        pltpu.sync_copy(xh.at[pl.ds(15 * ROWS_PT + ROWS_PT, ROWS_LAST - ROWS_PT)],
                        rows_v.at[pl.ds(0, ROWS_LAST - ROWS_PT)])
        do_rows(ROWS_PT, ROWS_LAST - ROWS_PT)

    pltpu.sync_copy(acc_v, out_hbm.at[c].at[s])


_segmax = functools.partial(
    pl.kernel, _segmax_body,
    out_type=jax.ShapeDtypeStruct((2, 16, G, DH), jnp.float32),
    mesh=_mesh,
    scratch_types=[
        pltpu.VMEM((G, DH), jnp.float32),
        pltpu.VMEM((SEG_CH, DH), jnp.float32),
        pltpu.VMEM((ROWS_LAST + 16, ), jnp.int32),
    ],
)()


def _final_body(p_ref, Wf_ref, bf_ref, o_ref):
    red0 = p_ref[0, 0]
    red1 = p_ref[1, 0]
    for s in range(1, 16):
        red0 = jnp.maximum(red0, p_ref[0, s])
        red1 = jnp.maximum(red1, p_ref[1, s])
    pooled = jnp.concatenate([red0, red1], axis=1)
    z = jnp.dot(pooled, Wf_ref[...], preferred_element_type=jnp.float32) \
        + bf_ref[...]
    o_ref[...] = 1.0 / (1.0 + jnp.exp(-z))


def _final(parts, Wf, bf):
    return pl.pallas_call(
        _final_body,
        out_shape=jax.ShapeDtypeStruct((G, NC), jnp.float32),
    )(parts, Wf, bf)


def kernel(data_base, edge_index_base, batch_base,
           W1a, b1a, W1b, b1b, W2a, b2a, W2b, b2b, W3a, b3a, W3b, b3b,
           g1, be1, g2, be2, g3, be3, Wf, bf):
    x = data_base.reshape(N, 2, DH).transpose(1, 0, 2)
    src2d = edge_index_base[0].reshape(NGRP, GRP, CHUNK)
    dst2d = edge_index_base[1].reshape(NGRP, GRP, CHUNK)
    neg = jnp.full((G, DH), -jnp.inf, jnp.float32)

    layers = [(W1a, b1a, W1b, b1b, g1, be1),
              (W2a, b2a, W2b, b2b, g2, be2),
              (W3a, b3a, W3b, b3b, g3, be3)]
    for Wa, ba, Wb, bb, g, be in layers:
        h = _scatter(x, src2d, dst2d)
        x = _mlp(h, Wa, ba.reshape(1, D), Wb, bb.reshape(1, D),
                 g.reshape(1, D), be.reshape(1, D))

    parts = _segmax(x, batch_base, neg)
    return _final(parts, Wf, bf.reshape(1, NC))


# speedup vs baseline: 1.2497x; 1.0496x over previous
import functools

import jax
import jax.numpy as jnp
from jax import lax
from jax.experimental import pallas as pl
from jax.experimental.pallas import tpu as pltpu
from jax.experimental.pallas import tpu_sc as plsc

N = 10000
E = 160000
D = 256
DH = 128
G = 128
NC = 40
BN_EPS = 1e-5

CHUNK = 125
GRP = 8
NGRP = E // (CHUNK * GRP)
GRP_PT = NGRP // 16
CH_T = GRP_PT * GRP
ROWS_PT = 624
ROWS_LAST = N - 15 * ROWS_PT
SEG_CH = 208

_mesh = plsc.VectorSubcoreMesh(core_axis_name="c", subcore_axis_name="s",
                               num_cores=2, num_subcores=16)


def _scatter_body(x_hbm, src_hbm, dst_hbm, out_hbm,
                  src_v, dst_v, rows_v, acc_s, gsem, isem):
    c = lax.axis_index("c")
    s = lax.axis_index("s")

    pltpu.sync_copy(x_hbm.at[c].at[pl.ds(s * ROWS_PT, ROWS_PT)],
                    acc_s.at[pl.ds(s * ROWS_PT, ROWS_PT)])

    @pl.when(s == 15)
    def _():
        pltpu.sync_copy(x_hbm.at[c].at[pl.ds(16 * ROWS_PT, ROWS_LAST - ROWS_PT)],
                        acc_s.at[pl.ds(16 * ROWS_PT, ROWS_LAST - ROWS_PT)])

    plsc.subcore_barrier()

    xh = x_hbm.at[c]
    gbase = s * GRP_PT

    def load_idx(g, a):
        pltpu.async_copy(src_hbm.at[gbase + g], src_v.at[a], isem.at[a])
        pltpu.async_copy(dst_hbm.at[gbase + g], dst_v.at[a], isem.at[a])

    def wait_idx(a):
        pltpu.make_async_copy(src_hbm.at[0], src_v.at[a], isem.at[a]).wait()
        pltpu.make_async_copy(dst_hbm.at[0], dst_v.at[a], isem.at[a]).wait()

    def gather(a, k, b):
        pltpu.async_copy(xh.at[src_v.at[a].at[k]], rows_v.at[b], gsem.at[b])

    def wait_gather(b):
        pltpu.make_async_copy(xh.at[src_v.at[0].at[0]], rows_v.at[b],
                              gsem.at[b]).wait()

    def scatter(a, k, b):
        pltpu.sync_copy(rows_v.at[b], acc_s.at[dst_v.at[a].at[k]], add=True)

    load_idx(0, 0)
    load_idx(1, 1)
    wait_idx(0)
    gather(0, 0, 0)
    gather(0, 1, 1)

    def group_body(g, aa, last, traced):
        for k in range(GRP):
            b = k % 2
            wait_gather(b)
            scatter(aa, k, b)
            if k < GRP - 2:
                gather(aa, k + 2, b)
            elif not last:
                if k == GRP - 2:
                    wait_idx(1 - aa)
                gather(1 - aa, k + 2 - GRP, b)
        if not last:
            if traced:
                @pl.when(g + 2 < GRP_PT)
                def _():
                    load_idx(g + 2, aa)
            elif g + 2 < GRP_PT:
                load_idx(g + 2, aa)

    def gpair(t, carry):
        g0 = 2 * t
        for aa in range(2):
            group_body(g0 + aa, aa, False, True)
        return carry

    if GRP_PT % 2 == 1:
        lax.fori_loop(0, (GRP_PT - 1) // 2, gpair, 0)
        group_body(GRP_PT - 1, 0, True, False)
    else:
        lax.fori_loop(0, (GRP_PT - 2) // 2, gpair, 0)
        group_body(GRP_PT - 2, 0, False, False)
        group_body(GRP_PT - 1, 1, True, False)

    plsc.subcore_barrier()

    pltpu.sync_copy(acc_s.at[pl.ds(s * ROWS_PT, ROWS_PT)],
                    out_hbm.at[c].at[pl.ds(s * ROWS_PT, ROWS_PT)])

    @pl.when(s == 15)
    def _():
        pltpu.sync_copy(acc_s.at[pl.ds(16 * ROWS_PT, ROWS_LAST - ROWS_PT)],
                        out_hbm.at[c].at[pl.ds(16 * ROWS_PT, ROWS_LAST - ROWS_PT)])


_scatter = functools.partial(
    pl.kernel, _scatter_body,
    out_type=jax.ShapeDtypeStruct((2, N, DH), jnp.float32),
    mesh=_mesh,
    scratch_types=[
        pltpu.VMEM((2, GRP, CHUNK), jnp.int32),
        pltpu.VMEM((2, GRP, CHUNK), jnp.int32),
        pltpu.VMEM((2, CHUNK, DH), jnp.float32),
        pltpu.VMEM_SHARED((N, DH), jnp.float32),
        pltpu.SemaphoreType.DMA((2,)),
        pltpu.SemaphoreType.DMA((2,)),
    ],
)()


NB = 2
BR = N // NB


def _mlp_body(h_ref, Wa_ref, ba_ref, Wb_ref, bb_ref, g_ref, be_ref,
              o_ref, y_scr, st_scr):
    p = pl.program_id(0)
    i = pl.program_id(1)

    @pl.when(p == 0)
    def _():
        hb = h_ref[...]
        h = jnp.concatenate([hb[0], hb[1]], axis=1)
        t = jnp.maximum(
            jnp.dot(h, Wa_ref[...], preferred_element_type=jnp.float32)
            + ba_ref[...], 0.0)
        y = jnp.maximum(
            jnp.dot(t, Wb_ref[...], preferred_element_type=jnp.float32)
            + bb_ref[...], 0.0)
        y_scr[pl.ds(i * BR, BR), :] = y

        @pl.when(i == 0)
        def _():
            st_scr[...] = jnp.zeros_like(st_scr)

        st_scr[0:1, :] += jnp.sum(y, axis=0, keepdims=True)
        st_scr[1:2, :] += jnp.sum(y * y, axis=0, keepdims=True)

    @pl.when(p == 1)
    def _():
        mu = st_scr[0:1, :] * (1.0 / N)
        var = st_scr[1:2, :] * (1.0 / N) - mu * mu
        scale = g_ref[...] * jax.lax.rsqrt(var + BN_EPS)
        shift = be_ref[...] - mu * scale
        y = y_scr[pl.ds(i * BR, BR), :] * scale + shift
        o_ref[0, :, :] = y[:, :DH]
        o_ref[1, :, :] = y[:, DH:]


def _mlp(h, Wa, ba, Wb, bb, g, be):
    wspec = pl.BlockSpec((D, D), lambda p, i: (0, 0))
    vspec = pl.BlockSpec((1, D), lambda p, i: (0, 0))
    return pl.pallas_call(
        _mlp_body,
        grid=(2, NB),
        in_specs=[
            pl.BlockSpec((2, BR, DH), lambda p, i: (0, jnp.where(p == 0, i, 0), 0)),
            wspec, vspec, wspec, vspec, vspec, vspec,
        ],
        out_specs=pl.BlockSpec((2, BR, DH), lambda p, i: (0, jnp.where(p == 1, i, 0), 0)),
        out_shape=jax.ShapeDtypeStruct((2, N, DH), jnp.float32),
        scratch_shapes=[
            pltpu.VMEM((N, D), jnp.float32),
            pltpu.VMEM((8, D), jnp.float32),
        ],
    )(h, Wa, ba, Wb, bb, g, be)


def _segmax_body(x_hbm, bat_hbm, neg_hbm, out_hbm, acc_v, rows_v, bat_v):
    c = lax.axis_index("c")
    s = lax.axis_index("s")

    pltpu.sync_copy(neg_hbm, acc_v)
    pltpu.sync_copy(bat_hbm.at[pl.ds(s * ROWS_PT, ROWS_PT)],
                    bat_v.at[pl.ds(0, ROWS_PT)])

    @pl.when(s == 15)
    def _():
        pltpu.sync_copy(bat_hbm.at[pl.ds(16 * ROWS_PT, ROWS_LAST - ROWS_PT)],
                        bat_v.at[pl.ds(ROWS_PT, ROWS_LAST - ROWS_PT)])

    xh = x_hbm.at[c]

    def do_rows(local_base, nrows):
        def block16(blk, carry2):
            rbase = blk * 16
            ids = bat_v[pl.ds(local_base + rbase, 16)]
            g0 = ids[0]

            @pl.when(g0 == ids[15])
            def _():
                for k in range(DH // 16):
                    sl = pl.ds(k * 16, 16)
                    m = rows_v[rbase, sl]
                    for r in range(1, 16):
                        m = jnp.maximum(m, rows_v[rbase + r, sl])
                    acc_v[g0, sl] = jnp.maximum(acc_v[g0, sl], m)

            @pl.when(g0 != ids[15])
            def _():
                def row(r, c3):
                    gidx = bat_v[pl.ds(local_base + rbase + r, 16)][0]
                    for k in range(DH // 16):
                        sl = pl.ds(k * 16, 16)
                        acc_v[gidx, sl] = jnp.maximum(acc_v[gidx, sl],
                                                      rows_v[rbase + r, sl])
                    return c3

                lax.fori_loop(0, 16, row, 0)

            return carry2

        lax.fori_loop(0, nrows // 16, block16, 0)

    def chunk(cb, carry):
        pltpu.sync_copy(xh.at[pl.ds(s * ROWS_PT + cb * SEG_CH, SEG_CH)], rows_v)
        do_rows(cb * SEG_CH, SEG_CH)
        return carry

    lax.fori_loop(0, ROWS_PT // SEG_CH, chunk, 0)

    @pl.when(s == 15)
    def _():
        pltpu.sync_copy(xh.at[pl.ds(15 * ROWS_PT + ROWS_PT, ROWS_LAST - ROWS_PT)],
                        rows_v.at[pl.ds(0, ROWS_LAST - ROWS_PT)])
        do_rows(ROWS_PT, ROWS_LAST - ROWS_PT)

    pltpu.sync_copy(acc_v, out_hbm.at[c].at[s])


_segmax = functools.partial(
    pl.kernel, _segmax_body,
    out_type=jax.ShapeDtypeStruct((2, 16, G, DH), jnp.float32),
    mesh=_mesh,
    scratch_types=[
        pltpu.VMEM((G, DH), jnp.float32),
        pltpu.VMEM((SEG_CH, DH), jnp.float32),
        pltpu.VMEM((ROWS_LAST + 16, ), jnp.int32),
    ],
)()


def _final_body(p_ref, Wf_ref, bf_ref, o_ref):
    red0 = p_ref[0, 0]
    red1 = p_ref[1, 0]
    for s in range(1, 16):
        red0 = jnp.maximum(red0, p_ref[0, s])
        red1 = jnp.maximum(red1, p_ref[1, s])
    pooled = jnp.concatenate([red0, red1], axis=1)
    z = jnp.dot(pooled, Wf_ref[...], preferred_element_type=jnp.float32) \
        + bf_ref[...]
    o_ref[...] = 1.0 / (1.0 + jnp.exp(-z))


def _final(parts, Wf, bf):
    return pl.pallas_call(
        _final_body,
        out_shape=jax.ShapeDtypeStruct((G, NC), jnp.float32),
    )(parts, Wf, bf)


def kernel(data_base, edge_index_base, batch_base,
           W1a, b1a, W1b, b1b, W2a, b2a, W2b, b2b, W3a, b3a, W3b, b3b,
           g1, be1, g2, be2, g3, be3, Wf, bf):
    x = data_base.reshape(N, 2, DH).transpose(1, 0, 2)
    src2d = edge_index_base[0].reshape(NGRP, GRP, CHUNK)
    dst2d = edge_index_base[1].reshape(NGRP, GRP, CHUNK)
    neg = jnp.full((G, DH), -jnp.inf, jnp.float32)

    layers = [(W1a, b1a, W1b, b1b, g1, be1),
              (W2a, b2a, W2b, b2b, g2, be2),
              (W3a, b3a, W3b, b3b, g3, be3)]
    for Wa, ba, Wb, bb, g, be in layers:
        h = _scatter(x, src2d, dst2d)
        x = _mlp(h, Wa, ba.reshape(1, D), Wb, bb.reshape(1, D),
                 g.reshape(1, D), be.reshape(1, D))

    parts = _segmax(x, batch_base, neg)
    return _final(parts, Wf, bf.reshape(1, NC))
